# baseline (speedup 1.0000x reference)
import functools

import jax
import jax.numpy as jnp
from jax import lax
from jax.experimental import pallas as pl
from jax.experimental.pallas import tpu as pltpu
from jax.experimental.pallas import tpu_sc as plsc

V = 10000
E = 320000
D = 128
NC = 2
NS = 16
NW = NC * NS
EPW = E // NW
K = 40
NCH = EPW // K
NB = 5
VP = 10240
RPT = VP // NS
L = 16
CR = VP // D

_sc_mesh = plsc.VectorSubcoreMesh(
    core_axis_name="c", subcore_axis_name="s", num_cores=NC, num_subcores=NS
)


@functools.partial(
    pl.kernel,
    out_type=(
        jax.ShapeDtypeStruct((NC, VP, D), jnp.float32),
        jax.ShapeDtypeStruct((NC, NS, CR, D), jnp.float32),
    ),
    mesh=_sc_mesh,
    compiler_params=pltpu.CompilerParams(
        use_tc_tiling_on_sc=False, needs_layout_passes=False
    ),
    scratch_types=[
        pltpu.VMEM((NB, K), jnp.int32),
        pltpu.VMEM((NCH, K), jnp.int32),
        [pltpu.VMEM((K, D), jnp.float32)] * NB,
        pltpu.VMEM((CR, D), jnp.float32),
        pltpu.VMEM_SHARED((VP, D), jnp.float32),
        [pltpu.SemaphoreType.DMA] * NB,
        [pltpu.SemaphoreType.DMA] * NB,
        [pltpu.SemaphoreType.DMA] * NB,
    ],
)
def _segment_sum_sc(feat, src, dst3, out, cnt_out,
                    srow, didx, rows, cnt, acc, isem, gsem, ssem):
    c = lax.axis_index("c")
    s = lax.axis_index("s")
    wid = c * NS + s
    base = wid * EPW

    pltpu.async_copy(dst3.at[wid], didx, gsem[0])
    z16 = jnp.zeros((L,), jnp.float32)

    def zero_buf(buf, nrow):
        def zb(i, carry):
            for u in range(8):
                buf[i, pl.ds(u * L, L)] = z16
            return carry
        lax.fori_loop(0, nrow, zb, 0)

    zero_buf(rows[0], K)
    zero_buf(cnt, CR)
    for r in range(RPT // K):
        sem = ssem[r % NB]
        pltpu.async_copy(rows[0], acc.at[pl.ds(s * RPT + r * K, K)], sem)
    for r in range(RPT // K):
        sem = ssem[r % NB]
        pltpu.make_async_copy(rows[0], acc.at[pl.ds(s * RPT + r * K, K)], sem).wait()
    pltpu.make_async_copy(dst3.at[wid], didx, gsem[0]).wait()
    plsc.subcore_barrier()

    def ifetch(ci, b):
        pltpu.async_copy(src.at[pl.ds(base + ci * K, K)], srow.at[b], isem[b])

    def iwait(ci, b):
        pltpu.make_async_copy(
            src.at[pl.ds(base + ci * K, K)], srow.at[b], isem[b]).wait()

    def gath(ci, b):
        pltpu.async_copy(feat.at[srow.at[b]], rows[b], gsem[b])

    def gwait(b):
        pltpu.make_async_copy(feat.at[srow.at[b]], rows[b], gsem[b]).wait()

    def sctr(ci, b):
        pltpu.async_copy(rows[b], acc.at[didx.at[ci]], ssem[b], add=True)

    def swait(b):
        pltpu.make_async_copy(rows[b], acc.at[didx.at[0]], ssem[b]).wait()

    ones = jnp.full((L,), 1.0, jnp.float32)
    lanes = lax.iota(jnp.int32, L)
    tailm = lanes >= 8

    def hist_group(idx16, gmask):
        hi = lax.shift_right_logical(idx16, 7)
        lo = lax.bitwise_and(idx16, 127)
        plsc.addupdate_scatter(cnt, [hi, lo], ones, mask=gmask)

    def hist(ci):
        hist_group(didx[ci, pl.ds(0, L)], None)
        hist_group(didx[ci, pl.ds(L, L)], None)
        hist_group(didx[ci, pl.ds(24, L)], tailm)

    for b in range(NB):
        ifetch(b, b)
    for b in range(3):
        iwait(b, b)
        gath(b, b)

    def step(ci, b, first):
        gwait(b)
        sctr(ci, b)
        hist(ci)

        @pl.when(ci + NB < NCH)
        def _():
            ifetch(ci + NB, b)

        @pl.when(ci + 3 < NCH)
        def _():
            bn = (b + 3) % NB
            if not first:
                swait(bn)
            iwait(ci + 3, bn)
            gath(ci + 3, bn)

    step(0, 0, True)
    step(1, 1, True)

    def body(t, carry):
        for j in range(NB):
            step(NB * t + j + 2, (j + 2) % NB, False)
        return carry

    lax.fori_loop(0, (NCH - 5) // NB, body, 0)
    for ci in range(NCH - 3, NCH):
        b = ci % NB
        gwait(b)
        sctr(ci, b)
        hist(ci)
    for b in range(NB):
        swait(b)

    pltpu.sync_copy(cnt, cnt_out.at[c, s])
    plsc.subcore_barrier()
    pltpu.sync_copy(acc.at[pl.ds(s * RPT, RPT)], out.at[c, pl.ds(s * RPT, RPT)])


BNC = 1024
BN = 1000


def _cnt_body(c_ref, o_ref):
    o_ref[...] = jnp.sum(c_ref[...], axis=0)[:, None]


def _cnt_reduce_tc(cnts):
    return pl.pallas_call(
        _cnt_body,
        grid=(VP // BNC,),
        in_specs=[pl.BlockSpec((NW, BNC), lambda i: (0, i))],
        out_specs=pl.BlockSpec((BNC, 1), lambda i: (i, 0)),
        out_shape=jax.ShapeDtypeStruct((VP, 1), jnp.float32),
    )(cnts)


def _tc_body(p_ref, c_ref, feat_ref, we_ref, wn_ref, b_ref, o_ref):
    s1 = p_ref[0] + p_ref[1]
    cnt = c_ref[...]
    inv = 1.0 / jnp.maximum(cnt, 1.0)
    msk = (cnt > 0.0).astype(jnp.float32)
    cmb = jnp.dot(wn_ref[...], we_ref[...], preferred_element_type=jnp.float32)
    h1 = lax.dot_general(s1 * inv, cmb[:, :D], (((1,), (1,)), ((), ())),
                         preferred_element_type=jnp.float32)
    h2 = lax.dot_general(feat_ref[...] * msk, cmb[:, D:], (((1,), (1,)), ((), ())),
                         preferred_element_type=jnp.float32)
    o_ref[...] = h1 + h2 + b_ref[...]


def _node_update_tc(partials, cntcol, feat, w_edge, w_node, b2):
    return pl.pallas_call(
        _tc_body,
        grid=(V // BN,),
        in_specs=[
            pl.BlockSpec((NC, BN, D), lambda i: (0, i, 0)),
            pl.BlockSpec((BN, 1), lambda i: (i, 0)),
            pl.BlockSpec((BN, D), lambda i: (i, 0)),
            pl.BlockSpec((D, 2 * D), lambda i: (0, 0)),
            pl.BlockSpec((D, D), lambda i: (0, 0)),
            pl.BlockSpec((1, D), lambda i: (0, 0)),
        ],
        out_specs=pl.BlockSpec((BN, D), lambda i: (i, 0)),
        out_shape=jax.ShapeDtypeStruct((V, D), jnp.float32),
    )(partials, cntcol, feat, w_edge, w_node, b2)


def kernel(feat, edge_index, W_edge, W_node, b_node):
    ei = edge_index.astype(jnp.int32)
    src = ei[0]
    dst3 = ei[1].reshape(NW, NCH, K)
    partials, cnts = _segment_sum_sc(feat, src, dst3)
    cntcol = _cnt_reduce_tc(cnts.reshape(NW, VP))
    return _node_update_tc(partials, cntcol, feat, W_edge, W_node,
                           b_node.reshape(1, D))


# speedup vs baseline: 1.0878x; 1.0878x over previous
import functools

import jax
import jax.numpy as jnp
from jax import lax
from jax.experimental import pallas as pl
from jax.experimental.pallas import tpu as pltpu
from jax.experimental.pallas import tpu_sc as plsc

V = 10000
E = 320000
D = 128
NC = 2
NS = 16
NW = NC * NS
EPW = E // NW
K = 40
NCH = EPW // K
NB = 6
VP = 10240
RPT = VP // NS
L = 16
CR = VP // D

_sc_mesh = plsc.VectorSubcoreMesh(
    core_axis_name="c", subcore_axis_name="s", num_cores=NC, num_subcores=NS
)


@functools.partial(
    pl.kernel,
    out_type=(
        jax.ShapeDtypeStruct((NC, VP, D), jnp.float32),
        jax.ShapeDtypeStruct((NC, NS, 2, K, D), jnp.float32),
    ),
    mesh=_sc_mesh,
    compiler_params=pltpu.CompilerParams(
        use_tc_tiling_on_sc=False, needs_layout_passes=False
    ),
    scratch_types=[
        pltpu.VMEM((NB, K), jnp.int32),
        pltpu.VMEM((NCH, K), jnp.int32),
        [pltpu.VMEM((K, D), jnp.float32)] * NB,
        pltpu.VMEM_SHARED((VP, D), jnp.float32),
        [pltpu.SemaphoreType.DMA] * NB,
        [pltpu.SemaphoreType.DMA] * NB,
        [pltpu.SemaphoreType.DMA] * NB,
    ],
)
def _segment_sum_sc(feat, src, dst3, out, cnt_out,
                    srow, didx, rows, acc, isem, gsem, ssem):
    c = lax.axis_index("c")
    s = lax.axis_index("s")
    wid = c * NS + s
    base = wid * EPW

    pltpu.async_copy(dst3.at[wid], didx, gsem[0])
    z16 = jnp.zeros((L,), jnp.float32)

    def zero_buf(buf):
        def zb(i, carry):
            for u in range(8):
                buf[i, pl.ds(u * L, L)] = z16
            return carry
        lax.fori_loop(0, K, zb, 0)

    def ifetch(ci, b):
        pltpu.async_copy(src.at[pl.ds(base + ci * K, K)], srow.at[b], isem[b])

    def iwait(ci, b):
        pltpu.make_async_copy(
            src.at[pl.ds(base + ci * K, K)], srow.at[b], isem[b]).wait()

    def gath(ci, b):
        pltpu.async_copy(feat.at[srow.at[b]], rows[b], gsem[b])

    def gwait(b):
        pltpu.make_async_copy(feat.at[srow.at[b]], rows[b], gsem[b]).wait()

    def sctr(ci, b):
        pltpu.async_copy(rows[b], acc.at[didx.at[ci]], ssem[b], add=True)

    def swait(b):
        pltpu.make_async_copy(rows[b], acc.at[didx.at[0]], ssem[b]).wait()

    for b in range(NB):
        ifetch(b, b)
    zero_buf(rows[4])
    for b in range(4):
        iwait(b, b)
        gath(b, b)
    for r in range(RPT // K):
        sem = ssem[r % NB]
        pltpu.async_copy(rows[4], acc.at[pl.ds(s * RPT + r * K, K)], sem)
    for r in range(RPT // K):
        sem = ssem[r % NB]
        pltpu.make_async_copy(rows[4], acc.at[pl.ds(s * RPT + r * K, K)], sem).wait()
    pltpu.make_async_copy(dst3.at[wid], didx, gsem[0]).wait()
    plsc.subcore_barrier()

    def step(ci, b, first):
        gwait(b)
        sctr(ci, b)

        @pl.when(ci + NB < NCH)
        def _():
            ifetch(ci + NB, b)

        @pl.when(ci + 4 < NCH)
        def _():
            bn = (b + 4) % NB
            if not first:
                swait(bn)
            iwait(ci + 4, bn)
            gath(ci + 4, bn)

    step(0, 0, True)
    step(1, 1, True)

    def body(t, carry):
        for j in range(NB):
            step(NB * t + j + 2, (j + 2) % NB, False)
        return carry

    lax.fori_loop(0, (NCH - 4) // NB, body, 0)
    gwait(2)
    sctr(NCH - 2, 2)
    gwait(3)
    sctr(NCH - 1, 3)
    for b in range(NB):
        swait(b)
    plsc.subcore_barrier()
    pltpu.async_copy(acc.at[pl.ds(s * RPT, RPT)], out.at[c, pl.ds(s * RPT, RPT)],
                     gsem[5])

    zero_buf(rows[0])
    zero_buf(rows[1])
    ones = jnp.full((L,), 1.0, jnp.float32)
    lanes = lax.iota(jnp.int32, L)
    tailm = lanes >= 8

    def hist_group(idx16, gmask):
        hi = lax.shift_right_logical(idx16, 7)
        lo = lax.bitwise_and(idx16, 127)
        mlow = hi < K
        if gmask is not None:
            mhi = gmask & jnp.logical_not(mlow)
            mlow = gmask & mlow
        else:
            mhi = jnp.logical_not(mlow)
        plsc.addupdate_scatter(rows[0], [hi, lo], ones, mask=mlow)
        plsc.addupdate_scatter(rows[1], [hi - K, lo], ones, mask=mhi)

    def hist_body(ci, carry):
        hist_group(didx[ci, pl.ds(0, L)], None)
        hist_group(didx[ci, pl.ds(L, L)], None)
        hist_group(didx[ci, pl.ds(24, L)], tailm)
        return carry

    lax.fori_loop(0, NCH, hist_body, 0)
    pltpu.sync_copy(rows[0], cnt_out.at[c, s, 0])
    pltpu.sync_copy(rows[1], cnt_out.at[c, s, 1])
    pltpu.make_async_copy(acc.at[pl.ds(s * RPT, RPT)],
                          out.at[c, pl.ds(s * RPT, RPT)], gsem[5]).wait()


BNC = 1024
BN = 1000


def _cnt_body(c_ref, o_ref):
    o_ref[...] = jnp.sum(c_ref[...], axis=0)[:, None]


def _cnt_reduce_tc(cnts):
    return pl.pallas_call(
        _cnt_body,
        grid=(VP // BNC,),
        in_specs=[pl.BlockSpec((NW, BNC), lambda i: (0, i))],
        out_specs=pl.BlockSpec((BNC, 1), lambda i: (i, 0)),
        out_shape=jax.ShapeDtypeStruct((VP, 1), jnp.float32),
    )(cnts)


def _tc_body(p_ref, c_ref, feat_ref, we_ref, wn_ref, b_ref, o_ref):
    s1 = p_ref[0] + p_ref[1]
    cnt = c_ref[...]
    inv = 1.0 / jnp.maximum(cnt, 1.0)
    msk = (cnt > 0.0).astype(jnp.float32)
    cmb = jnp.dot(wn_ref[...], we_ref[...], preferred_element_type=jnp.float32)
    h1 = lax.dot_general(s1 * inv, cmb[:, :D], (((1,), (1,)), ((), ())),
                         preferred_element_type=jnp.float32)
    h2 = lax.dot_general(feat_ref[...] * msk, cmb[:, D:], (((1,), (1,)), ((), ())),
                         preferred_element_type=jnp.float32)
    o_ref[...] = h1 + h2 + b_ref[...]


def _node_update_tc(partials, cntcol, feat, w_edge, w_node, b2):
    return pl.pallas_call(
        _tc_body,
        grid=(V // BN,),
        in_specs=[
            pl.BlockSpec((NC, BN, D), lambda i: (0, i, 0)),
            pl.BlockSpec((BN, 1), lambda i: (i, 0)),
            pl.BlockSpec((BN, D), lambda i: (i, 0)),
            pl.BlockSpec((D, 2 * D), lambda i: (0, 0)),
            pl.BlockSpec((D, D), lambda i: (0, 0)),
            pl.BlockSpec((1, D), lambda i: (0, 0)),
        ],
        out_specs=pl.BlockSpec((BN, D), lambda i: (i, 0)),
        out_shape=jax.ShapeDtypeStruct((V, D), jnp.float32),
    )(partials, cntcol, feat, w_edge, w_node, b2)


def kernel(feat, edge_index, W_edge, W_node, b_node):
    ei = edge_index.astype(jnp.int32)
    src = ei[0]
    dst3 = ei[1].reshape(NW, NCH, K)
    partials, cnts = _segment_sum_sc(feat, src, dst3)
    cntcol = _cnt_reduce_tc(cnts.reshape(NW, VP))
    return _node_update_tc(partials, cntcol, feat, W_edge, W_node,
                           b_node.reshape(1, D))
